# Initial kernel scaffold; baseline (speedup 1.0000x reference)
import jax
import jax.numpy as jnp
from jax.experimental import pallas as pl


def kernel(inputs, edges_p, max_edges):
    raise NotImplementedError("write your pallas kernel here")


# speedup vs baseline: 2.1560x; 2.1560x over previous
import functools

import jax
import jax.numpy as jnp
from jax import lax
from jax.experimental import pallas as pl
from jax.experimental.pallas import tpu as pltpu
from jax.experimental.pallas import tpu_sc as plsc

BATCH = 32
GS = 25000
K = 2048
PAD = 25088
CHUNKS = 196
VREGS = PAD // 16
NB = 4


def _sc_select_build():
    mesh = plsc.VectorSubcoreMesh(core_axis_name="c", subcore_axis_name="s")

    @functools.partial(
        pl.kernel,
        mesh=mesh,
        compiler_params=pltpu.CompilerParams(needs_layout_passes=False),
        out_type=[
            jax.ShapeDtypeStruct((BATCH, K), jnp.int32),
            jax.ShapeDtypeStruct((BATCH, K), jnp.int32),
            jax.ShapeDtypeStruct((BATCH, K), jnp.int32),
            jax.ShapeDtypeStruct((BATCH, 16), jnp.float32),
        ],
        scratch_types=[
            pltpu.VMEM((CHUNKS, 128), jnp.int32),
            pltpu.VMEM((PAD,), jnp.float32),
            pltpu.VMEM((PAD,), jnp.float32),
            pltpu.VMEM((PAD,), jnp.int32),
            pltpu.VMEM((K + 16,), jnp.int32),
            pltpu.VMEM((K + 16,), jnp.int32),
            pltpu.VMEM((K + 16,), jnp.int32),
            pltpu.VMEM((16,), jnp.float32),
            pltpu.SemaphoreType.DMA,
        ],
    )
    def sc_select(eid_hbm, g_hbm, table_hbm, okey, oidx, oeid, osum,
                  eid2d, g_v, p_v, ukey_v, ck_v, ci_v, ce_v, s_v, sem):
        t = lax.axis_index("s") * 2 + lax.axis_index("c")

        pltpu.sync_copy(eid_hbm.at[t], eid2d)
        pltpu.sync_copy(g_hbm.at[t], g_v)

        def gblk(b, carry):
            base = b * NB
            for j in range(NB):
                pltpu.async_copy(
                    table_hbm.at[eid2d.at[base + j]],
                    p_v.at[pl.ds((base + j) * 128, 128)], sem)
            for j in range(NB):
                pltpu.make_async_copy(
                    table_hbm.at[eid2d.at[base + j]],
                    p_v.at[pl.ds((base + j) * 128, 128)], sem).wait()
            return carry
        lax.fori_loop(0, CHUNKS // NB, gblk, 0)

        IMIN = jnp.int32(-2147483648)
        def kbody(j, carry):
            y = p_v[pl.ds(j * 16, 16)] + g_v[pl.ds(j * 16, 16)]
            si = lax.bitcast_convert_type(y, jnp.int32)
            sk = jnp.where(si < 0, si ^ jnp.int32(0x7FFFFFFF), si)
            valid = (j * 16 + lax.iota(jnp.int32, 16)) < GS
            ukey_v[pl.ds(j * 16, 16)] = jnp.where(valid, sk, IMIN)
            return carry
        lax.fori_loop(0, VREGS, kbody, 0)

        lane15 = jnp.full((16,), 15, jnp.int32)
        def mx_body(j, acc):
            return jnp.maximum(acc, ukey_v[pl.ds(j * 16, 16)])
        mvec = lax.fori_loop(0, VREGS, mx_body, jnp.full((16,), IMIN, jnp.int32))
        mtot = plsc.cummax(mvec).at[lane15].get(mode="promise_in_bounds")
        msk = jnp.where(mtot < 0, mtot ^ jnp.int32(0x7FFFFFFF), mtot)
        My = lax.bitcast_convert_type(msk, jnp.float32)
        def es_body(j, acc):
            y = p_v[pl.ds(j * 16, 16)] + g_v[pl.ds(j * 16, 16)]
            ev = jnp.exp(y - My)
            valid = (j * 16 + lax.iota(jnp.int32, 16)) < GS
            return acc + jnp.where(valid, ev, jnp.float32(0))
        svec = lax.fori_loop(0, VREGS, es_body, jnp.zeros((16,), jnp.float32))
        s_v[pl.ds(0, 16)] = plsc.cumsum(svec).at[lane15].get(
            mode="promise_in_bounds")

        def bitstep(bi, Tu):
            cand_u = Tu | (jnp.int32(1) << (jnp.int32(31) - bi))
            cand_s = cand_u ^ IMIN
            def cnt_body(j, cnt):
                m = ukey_v[pl.ds(j * 16, 16)] >= cand_s
                return cnt + jnp.sum(m.astype(jnp.int32))
            cnt = lax.fori_loop(0, VREGS, cnt_body, jnp.int32(0))
            return jnp.where(cnt >= K, cand_u, Tu)
        Tu = lax.fori_loop(0, 32, bitstep, jnp.int32(0))
        T = Tu ^ IMIN

        def cg_body(j, cnt):
            m = ukey_v[pl.ds(j * 16, 16)] > T
            return cnt + jnp.sum(m.astype(jnp.int32))
        cg = lax.fori_loop(0, VREGS, cg_body, jnp.int32(0))
        need = K - cg

        def comp_body(j, carry):
            w, teq = carry
            skv = ukey_v[pl.ds(j * 16, 16)]
            gt = skv > T
            eq = skv == T
            eqc = jnp.cumsum(eq.astype(jnp.int32))
            sel = jnp.logical_or(gt, jnp.logical_and(eq, (teq + eqc) <= need))
            idxv = j * 16 + lax.iota(jnp.int32, 16)
            eidv = eid2d[j // 8, pl.ds((j % 8) * 16, 16)]
            plsc.store_compressed(ck_v.at[pl.ds(w, 16)], skv, mask=sel)
            plsc.store_compressed(ci_v.at[pl.ds(w, 16)], idxv, mask=sel)
            plsc.store_compressed(ce_v.at[pl.ds(w, 16)], eidv, mask=sel)
            return (w + jnp.sum(sel.astype(jnp.int32)),
                    teq + jnp.sum(eq.astype(jnp.int32)))
        lax.fori_loop(0, VREGS, comp_body, (jnp.int32(0), jnp.int32(0)))

        pltpu.sync_copy(ck_v.at[pl.ds(0, K)], okey.at[t])
        pltpu.sync_copy(ci_v.at[pl.ds(0, K)], oidx.at[t])
        pltpu.sync_copy(ce_v.at[pl.ds(0, K)], oeid.at[t])
        pltpu.sync_copy(s_v, osum.at[t])

    return sc_select


_sc_select = _sc_select_build()


def _tc_sort_body(ck_ref, ci_ref, ce_ref, s_ref, oe_ref, ones_ref):
    sk = ck_ref[...]
    i = ci_ref[...]
    e = ce_ref[...]
    si = jnp.where(sk < 0, sk ^ jnp.int32(0x7FFFFFFF), sk)
    y = lax.bitcast_convert_type(si, jnp.float32)
    M = jnp.max(y, axis=1, keepdims=True)
    k = jnp.exp(y - M) / s_ref[:, 0:1]
    lane = lax.broadcasted_iota(jnp.int32, (BATCH, K), 1)
    s = 2
    while s <= K:
        d = s // 2
        while d >= 1:
            bit = (lane & d) != 0
            dirup = (lane & s) == 0
            def par(x, d=d, bit=bit):
                return jnp.where(bit, jnp.roll(x, d, axis=1),
                                 jnp.roll(x, -d, axis=1))
            pk, pi, pe = par(k), par(i), par(e)
            self_first = (k > pk) | ((k == pk) & (i < pi))
            take_self = self_first == (jnp.logical_not(bit) == dirup)
            k = jnp.where(take_self, k, pk)
            i = jnp.where(take_self, i, pi)
            e = jnp.where(take_self, e, pe)
            d //= 2
        s *= 2
    oe_ref[...] = e
    ones_ref[...] = jnp.ones_like(ones_ref)


def _tc_sort(ck, ci, ce, s):
    return pl.pallas_call(
        _tc_sort_body,
        out_shape=[
            jax.ShapeDtypeStruct((BATCH, K), jnp.int32),
            jax.ShapeDtypeStruct((BATCH * K,), jnp.float32),
        ],
    )(ck, ci, ce, s)


def kernel(inputs, edges_p, max_edges):
    edge_id = inputs[:, 1].reshape(BATCH, GS)
    eid_pad = jnp.zeros((BATCH, PAD), jnp.int32).at[:, :GS].set(edge_id)
    eid3 = eid_pad.reshape(BATCH, CHUNKS, 128)
    key42 = jax.random.key(42)
    eps = 1e-20
    gs = [-(jnp.log(-jnp.log(jax.random.uniform(
        jax.random.fold_in(key42, t), (GS,), dtype=jnp.float32) + eps) + eps))
        for t in range(BATCH)]
    g_pad = jnp.zeros((BATCH, PAD), jnp.float32).at[:, :GS].set(jnp.stack(gs))

    ck, ci, ce, ssum = _sc_select(eid3, g_pad, edges_p)
    eid_sorted, ones = _tc_sort(ck, ci, ce, ssum)

    eg = jnp.repeat(jnp.arange(BATCH, dtype=jnp.int32), K)
    outputs = jnp.stack([eg, eid_sorted.reshape(-1)], axis=1)
    return outputs, ones


# speedup vs baseline: 2.2426x; 1.0402x over previous
import functools

import jax
import jax.numpy as jnp
from jax import lax
from jax.experimental import pallas as pl
from jax.experimental.pallas import tpu as pltpu
from jax.experimental.pallas import tpu_sc as plsc

BATCH = 32
GS = 25000
K = 2048
PAD = 25088
CHUNKS = 196
VREGS = PAD // 16
NB = 14


def _sc_select_build():
    mesh = plsc.VectorSubcoreMesh(core_axis_name="c", subcore_axis_name="s")

    @functools.partial(
        pl.kernel,
        mesh=mesh,
        compiler_params=pltpu.CompilerParams(needs_layout_passes=False),
        out_type=[
            jax.ShapeDtypeStruct((BATCH, K), jnp.int32),
            jax.ShapeDtypeStruct((BATCH, K), jnp.int32),
            jax.ShapeDtypeStruct((BATCH, K), jnp.int32),
            jax.ShapeDtypeStruct((BATCH, 16), jnp.float32),
        ],
        scratch_types=[
            pltpu.VMEM((CHUNKS, 128), jnp.int32),
            pltpu.VMEM((PAD,), jnp.float32),
            pltpu.VMEM((PAD,), jnp.float32),
            pltpu.VMEM((PAD,), jnp.int32),
            pltpu.VMEM((K + 16,), jnp.int32),
            pltpu.VMEM((K + 16,), jnp.int32),
            pltpu.VMEM((K + 16,), jnp.int32),
            pltpu.VMEM((16,), jnp.float32),
            pltpu.SemaphoreType.DMA,
        ],
    )
    def sc_select(eid_hbm, g_hbm, table_hbm, okey, oidx, oeid, osum,
                  eid2d, g_v, p_v, ukey_v, ck_v, ci_v, ce_v, s_v, sem):
        t = lax.axis_index("s") * 2 + lax.axis_index("c")

        pltpu.sync_copy(eid_hbm.at[t], eid2d)
        pltpu.sync_copy(g_hbm.at[t], g_v)

        def gblk(b, carry):
            base = b * NB
            for j in range(NB):
                pltpu.async_copy(
                    table_hbm.at[eid2d.at[base + j]],
                    p_v.at[pl.ds((base + j) * 128, 128)], sem)
            for j in range(NB):
                pltpu.make_async_copy(
                    table_hbm.at[eid2d.at[base + j]],
                    p_v.at[pl.ds((base + j) * 128, 128)], sem).wait()
            return carry
        lax.fori_loop(0, CHUNKS // NB, gblk, 0)

        IMIN = jnp.int32(-2147483648)
        def kbody(j, carry):
            y = p_v[pl.ds(j * 16, 16)] + g_v[pl.ds(j * 16, 16)]
            si = lax.bitcast_convert_type(y, jnp.int32)
            sk = jnp.where(si < 0, si ^ jnp.int32(0x7FFFFFFF), si)
            valid = (j * 16 + lax.iota(jnp.int32, 16)) < GS
            ukey_v[pl.ds(j * 16, 16)] = jnp.where(valid, sk, IMIN)
            return carry
        lax.fori_loop(0, VREGS, kbody, 0)

        lane15 = jnp.full((16,), 15, jnp.int32)
        def mx_body(j, acc):
            return jnp.maximum(acc, ukey_v[pl.ds(j * 16, 16)])
        mvec = lax.fori_loop(0, VREGS, mx_body, jnp.full((16,), IMIN, jnp.int32))
        mtot = plsc.cummax(mvec).at[lane15].get(mode="promise_in_bounds")
        msk = jnp.where(mtot < 0, mtot ^ jnp.int32(0x7FFFFFFF), mtot)
        My = lax.bitcast_convert_type(msk, jnp.float32)
        def es_body(j, acc):
            y = p_v[pl.ds(j * 16, 16)] + g_v[pl.ds(j * 16, 16)]
            ev = jnp.exp(y - My)
            valid = (j * 16 + lax.iota(jnp.int32, 16)) < GS
            return acc + jnp.where(valid, ev, jnp.float32(0))
        svec = lax.fori_loop(0, VREGS, es_body, jnp.zeros((16,), jnp.float32))
        s_v[pl.ds(0, 16)] = plsc.cumsum(svec).at[lane15].get(
            mode="promise_in_bounds")

        def bitstep(bi, Tu):
            cand_u = Tu | (jnp.int32(1) << (jnp.int32(31) - bi))
            cand_s = cand_u ^ IMIN
            def cnt_body(j, cvec):
                m = ukey_v[pl.ds(j * 16, 16)] >= cand_s
                return cvec + jnp.where(m, jnp.int32(1), jnp.int32(0))
            cvec = lax.fori_loop(0, VREGS, cnt_body, jnp.zeros((16,), jnp.int32))
            cnt = plsc.cumsum(cvec).at[lane15].get(mode="promise_in_bounds")
            return jnp.where(cnt >= K, cand_u, Tu)
        Tu = lax.fori_loop(0, 32, bitstep, jnp.full((16,), 0, jnp.int32))
        T = Tu ^ IMIN

        def cg_body(j, cvec):
            m = ukey_v[pl.ds(j * 16, 16)] > T
            return cvec + jnp.where(m, jnp.int32(1), jnp.int32(0))
        cgv = lax.fori_loop(0, VREGS, cg_body, jnp.zeros((16,), jnp.int32))
        cg = plsc.cumsum(cgv).at[lane15].get(mode="promise_in_bounds")
        need = K - cg

        def comp_body(j, carry):
            w, teq = carry
            skv = ukey_v[pl.ds(j * 16, 16)]
            gt = skv > T
            eq = skv == T
            eqc = jnp.cumsum(eq.astype(jnp.int32))
            sel = jnp.logical_or(gt, jnp.logical_and(eq, (teq + eqc) <= need))
            idxv = j * 16 + lax.iota(jnp.int32, 16)
            eidv = eid2d[j // 8, pl.ds((j % 8) * 16, 16)]
            plsc.store_compressed(ck_v.at[pl.ds(w, 16)], skv, mask=sel)
            plsc.store_compressed(ci_v.at[pl.ds(w, 16)], idxv, mask=sel)
            plsc.store_compressed(ce_v.at[pl.ds(w, 16)], eidv, mask=sel)
            return (w + jnp.sum(sel.astype(jnp.int32)),
                    teq + jnp.sum(eq.astype(jnp.int32)))
        lax.fori_loop(0, VREGS, comp_body, (jnp.int32(0), jnp.int32(0)))

        pltpu.sync_copy(ck_v.at[pl.ds(0, K)], okey.at[t])
        pltpu.sync_copy(ci_v.at[pl.ds(0, K)], oidx.at[t])
        pltpu.sync_copy(ce_v.at[pl.ds(0, K)], oeid.at[t])
        pltpu.sync_copy(s_v, osum.at[t])

    return sc_select


_sc_select = _sc_select_build()


def _tc_sort_body(ck_ref, ci_ref, ce_ref, s_ref, oe_ref, ones_ref):
    sk = ck_ref[...]
    i = ci_ref[...]
    e = ce_ref[...]
    si = jnp.where(sk < 0, sk ^ jnp.int32(0x7FFFFFFF), sk)
    y = lax.bitcast_convert_type(si, jnp.float32)
    M = jnp.max(y, axis=1, keepdims=True)
    k = jnp.exp(y - M) / s_ref[:, 0:1]
    lane = lax.broadcasted_iota(jnp.int32, (BATCH, K), 1)
    s = 2
    while s <= K:
        d = s // 2
        while d >= 1:
            bit = (lane & d) != 0
            dirup = (lane & s) == 0
            def par(x, d=d, bit=bit):
                return jnp.where(bit, jnp.roll(x, d, axis=1),
                                 jnp.roll(x, -d, axis=1))
            pk, pi, pe = par(k), par(i), par(e)
            self_first = (k > pk) | ((k == pk) & (i < pi))
            take_self = self_first == (jnp.logical_not(bit) == dirup)
            k = jnp.where(take_self, k, pk)
            i = jnp.where(take_self, i, pi)
            e = jnp.where(take_self, e, pe)
            d //= 2
        s *= 2
    oe_ref[...] = e
    ones_ref[...] = jnp.ones_like(ones_ref)


def _tc_sort(ck, ci, ce, s):
    return pl.pallas_call(
        _tc_sort_body,
        out_shape=[
            jax.ShapeDtypeStruct((BATCH, K), jnp.int32),
            jax.ShapeDtypeStruct((BATCH * K,), jnp.float32),
        ],
    )(ck, ci, ce, s)


def kernel(inputs, edges_p, max_edges):
    edge_id = inputs[:, 1].reshape(BATCH, GS)
    eid_pad = jnp.zeros((BATCH, PAD), jnp.int32).at[:, :GS].set(edge_id)
    eid3 = eid_pad.reshape(BATCH, CHUNKS, 128)
    key42 = jax.random.key(42)
    eps = 1e-20
    gs = [-(jnp.log(-jnp.log(jax.random.uniform(
        jax.random.fold_in(key42, t), (GS,), dtype=jnp.float32) + eps) + eps))
        for t in range(BATCH)]
    g_pad = jnp.zeros((BATCH, PAD), jnp.float32).at[:, :GS].set(jnp.stack(gs))

    ck, ci, ce, ssum = _sc_select(eid3, g_pad, edges_p)
    eid_sorted, ones = _tc_sort(ck, ci, ce, ssum)

    eg = jnp.repeat(jnp.arange(BATCH, dtype=jnp.int32), K)
    outputs = jnp.stack([eg, eid_sorted.reshape(-1)], axis=1)
    return outputs, ones


# speedup vs baseline: 3.0713x; 1.3695x over previous
import functools

import jax
import jax.numpy as jnp
from jax import lax
from jax.experimental import pallas as pl
from jax.experimental.pallas import tpu as pltpu
from jax.experimental.pallas import tpu_sc as plsc

BATCH = 32
GS = 25000
K = 2048
PAD = 25088
CHUNKS = 196
VREGS = PAD // 16
NB = 14


def _sc_select_build():
    mesh = plsc.VectorSubcoreMesh(core_axis_name="c", subcore_axis_name="s")

    @functools.partial(
        pl.kernel,
        mesh=mesh,
        compiler_params=pltpu.CompilerParams(needs_layout_passes=False),
        out_type=[
            jax.ShapeDtypeStruct((BATCH, K), jnp.int32),
            jax.ShapeDtypeStruct((BATCH, K), jnp.int32),
            jax.ShapeDtypeStruct((BATCH, K), jnp.int32),
            jax.ShapeDtypeStruct((BATCH, 16), jnp.float32),
        ],
        scratch_types=[
            pltpu.VMEM((CHUNKS, 128), jnp.int32),
            pltpu.VMEM((PAD,), jnp.float32),
            pltpu.VMEM((PAD,), jnp.float32),
            pltpu.VMEM((PAD,), jnp.int32),
            pltpu.VMEM((K + 16,), jnp.int32),
            pltpu.VMEM((K + 16,), jnp.int32),
            pltpu.VMEM((K + 16,), jnp.int32),
            pltpu.VMEM((16,), jnp.float32),
            pltpu.SemaphoreType.DMA,
        ],
    )
    def sc_select(eid_hbm, g_hbm, table_hbm, okey, oidx, oeid, osum,
                  eid2d, g_v, p_v, ukey_v, ck_v, ci_v, ce_v, s_v, sem):
        t = lax.axis_index("s") * 2 + lax.axis_index("c")

        pltpu.sync_copy(eid_hbm.at[t], eid2d)
        pltpu.sync_copy(g_hbm.at[t], g_v)

        def gblk(b, carry):
            base = b * NB
            for j in range(NB):
                pltpu.async_copy(
                    table_hbm.at[eid2d.at[base + j]],
                    p_v.at[pl.ds((base + j) * 128, 128)], sem)
            for j in range(NB):
                pltpu.make_async_copy(
                    table_hbm.at[eid2d.at[base + j]],
                    p_v.at[pl.ds((base + j) * 128, 128)], sem).wait()
            return carry
        lax.fori_loop(0, CHUNKS // NB, gblk, 0)

        IMIN = jnp.int32(-2147483648)
        def kbody(j, carry):
            for u in range(4):
                b0 = (j * 4 + u) * 16
                y = p_v[pl.ds(b0, 16)] + g_v[pl.ds(b0, 16)]
                si = lax.bitcast_convert_type(y, jnp.int32)
                sk = jnp.where(si < 0, si ^ jnp.int32(0x7FFFFFFF), si)
                valid = (b0 + lax.iota(jnp.int32, 16)) < GS
                ukey_v[pl.ds(b0, 16)] = jnp.where(valid, sk, IMIN)
            return carry
        lax.fori_loop(0, VREGS // 4, kbody, 0)

        lane15 = jnp.full((16,), 15, jnp.int32)
        def mx_body(j, acc):
            for u in range(4):
                acc = jnp.maximum(acc, ukey_v[pl.ds((j * 4 + u) * 16, 16)])
            return acc
        mvec = lax.fori_loop(0, VREGS // 4, mx_body, jnp.full((16,), IMIN, jnp.int32))
        mtot = plsc.cummax(mvec).at[lane15].get(mode="promise_in_bounds")
        msk = jnp.where(mtot < 0, mtot ^ jnp.int32(0x7FFFFFFF), mtot)
        My = lax.bitcast_convert_type(msk, jnp.float32)
        def es_body(j, acc):
            for u in range(4):
                b0 = (j * 4 + u) * 16
                y = p_v[pl.ds(b0, 16)] + g_v[pl.ds(b0, 16)]
                ev = jnp.exp(y - My)
                valid = (b0 + lax.iota(jnp.int32, 16)) < GS
                acc = acc + jnp.where(valid, ev, jnp.float32(0))
            return acc
        svec = lax.fori_loop(0, VREGS // 4, es_body, jnp.zeros((16,), jnp.float32))
        s_v[pl.ds(0, 16)] = plsc.cumsum(svec).at[lane15].get(
            mode="promise_in_bounds")

        def bitstep(bi, Tu):
            cand_u = Tu | (jnp.int32(1) << (jnp.int32(31) - bi))
            cand_s = cand_u ^ IMIN
            def cnt_body(j, cvec):
                for u in range(4):
                    m = ukey_v[pl.ds((j * 4 + u) * 16, 16)] >= cand_s
                    cvec = cvec + jnp.where(m, jnp.int32(1), jnp.int32(0))
                return cvec
            cvec = lax.fori_loop(0, VREGS // 4, cnt_body, jnp.zeros((16,), jnp.int32))
            cnt = plsc.cumsum(cvec).at[lane15].get(mode="promise_in_bounds")
            return jnp.where(cnt >= K, cand_u, Tu)
        Tu = lax.fori_loop(0, 32, bitstep, jnp.full((16,), 0, jnp.int32))
        T = Tu ^ IMIN

        def cg_body(j, cvec):
            for u in range(4):
                m = ukey_v[pl.ds((j * 4 + u) * 16, 16)] > T
                cvec = cvec + jnp.where(m, jnp.int32(1), jnp.int32(0))
            return cvec
        cgv = lax.fori_loop(0, VREGS // 4, cg_body, jnp.zeros((16,), jnp.int32))
        cg = plsc.cumsum(cgv).at[lane15].get(mode="promise_in_bounds")
        need = K - cg

        def comp_body(j, carry):
            w, teq = carry
            skv = ukey_v[pl.ds(j * 16, 16)]
            gt = skv > T
            eq = skv == T
            eqc = jnp.cumsum(eq.astype(jnp.int32))
            sel = jnp.logical_or(gt, jnp.logical_and(eq, (teq + eqc) <= need))
            idxv = j * 16 + lax.iota(jnp.int32, 16)
            eidv = eid2d[j // 8, pl.ds((j % 8) * 16, 16)]
            plsc.store_compressed(ck_v.at[pl.ds(w, 16)], skv, mask=sel)
            plsc.store_compressed(ci_v.at[pl.ds(w, 16)], idxv, mask=sel)
            plsc.store_compressed(ce_v.at[pl.ds(w, 16)], eidv, mask=sel)
            return (w + jnp.sum(sel.astype(jnp.int32)),
                    teq + jnp.sum(eq.astype(jnp.int32)))
        lax.fori_loop(0, VREGS, comp_body, (jnp.int32(0), jnp.int32(0)))

        pltpu.sync_copy(ck_v.at[pl.ds(0, K)], okey.at[t])
        pltpu.sync_copy(ci_v.at[pl.ds(0, K)], oidx.at[t])
        pltpu.sync_copy(ce_v.at[pl.ds(0, K)], oeid.at[t])
        pltpu.sync_copy(s_v, osum.at[t])

    return sc_select


_sc_select = _sc_select_build()


def _tc_sort_body(ck_ref, ci_ref, ce_ref, s_ref, oe_ref, ones_ref):
    sk = ck_ref[...]
    i = ci_ref[...]
    e = ce_ref[...]
    si = jnp.where(sk < 0, sk ^ jnp.int32(0x7FFFFFFF), sk)
    y = lax.bitcast_convert_type(si, jnp.float32)
    M = jnp.max(y, axis=1, keepdims=True)
    k = jnp.exp(y - M) / s_ref[:, 0:1]
    lane = lax.broadcasted_iota(jnp.int32, (BATCH, K), 1)
    s = 2
    while s <= K:
        d = s // 2
        while d >= 1:
            bit = (lane & d) != 0
            dirup = (lane & s) == 0
            def par(x, d=d, bit=bit):
                return jnp.where(bit, jnp.roll(x, d, axis=1),
                                 jnp.roll(x, -d, axis=1))
            pk, pi, pe = par(k), par(i), par(e)
            self_first = (k > pk) | ((k == pk) & (i < pi))
            take_self = self_first == (jnp.logical_not(bit) == dirup)
            k = jnp.where(take_self, k, pk)
            i = jnp.where(take_self, i, pi)
            e = jnp.where(take_self, e, pe)
            d //= 2
        s *= 2
    oe_ref[...] = e
    ones_ref[...] = jnp.ones_like(ones_ref)


def _tc_sort(ck, ci, ce, s):
    return pl.pallas_call(
        _tc_sort_body,
        out_shape=[
            jax.ShapeDtypeStruct((BATCH, K), jnp.int32),
            jax.ShapeDtypeStruct((BATCH * K,), jnp.float32),
        ],
    )(ck, ci, ce, s)


def kernel(inputs, edges_p, max_edges):
    edge_id = inputs[:, 1].reshape(BATCH, GS)
    eid_pad = jnp.zeros((BATCH, PAD), jnp.int32).at[:, :GS].set(edge_id)
    eid3 = eid_pad.reshape(BATCH, CHUNKS, 128)
    key42 = jax.random.key(42)
    eps = 1e-20
    gs = [-(jnp.log(-jnp.log(jax.random.uniform(
        jax.random.fold_in(key42, t), (GS,), dtype=jnp.float32) + eps) + eps))
        for t in range(BATCH)]
    g_pad = jnp.zeros((BATCH, PAD), jnp.float32).at[:, :GS].set(jnp.stack(gs))

    ck, ci, ce, ssum = _sc_select(eid3, g_pad, edges_p)
    eid_sorted, ones = _tc_sort(ck, ci, ce, ssum)

    eg = jnp.repeat(jnp.arange(BATCH, dtype=jnp.int32), K)
    outputs = jnp.stack([eg, eid_sorted.reshape(-1)], axis=1)
    return outputs, ones


# speedup vs baseline: 3.2024x; 1.0427x over previous
import functools

import jax
import jax.numpy as jnp
from jax import lax
from jax.experimental import pallas as pl
from jax.experimental.pallas import tpu as pltpu
from jax.experimental.pallas import tpu_sc as plsc

BATCH = 32
GS = 25000
K = 2048
PAD = 25088
CHUNKS = 196
VREGS = PAD // 16
NB = 14


def _sc_select_build():
    mesh = plsc.VectorSubcoreMesh(core_axis_name="c", subcore_axis_name="s")

    @functools.partial(
        pl.kernel,
        mesh=mesh,
        compiler_params=pltpu.CompilerParams(needs_layout_passes=False),
        out_type=[
            jax.ShapeDtypeStruct((BATCH, K), jnp.int32),
            jax.ShapeDtypeStruct((BATCH, K), jnp.int32),
            jax.ShapeDtypeStruct((BATCH, K), jnp.int32),
            jax.ShapeDtypeStruct((BATCH, 16), jnp.float32),
        ],
        scratch_types=[
            pltpu.VMEM((CHUNKS, 128), jnp.int32),
            pltpu.VMEM((PAD,), jnp.float32),
            pltpu.VMEM((PAD,), jnp.float32),
            pltpu.VMEM((PAD,), jnp.int32),
            pltpu.VMEM((K + 16,), jnp.int32),
            pltpu.VMEM((K + 16,), jnp.int32),
            pltpu.VMEM((K + 16,), jnp.int32),
            pltpu.VMEM((16,), jnp.float32),
            pltpu.SemaphoreType.DMA,
        ],
    )
    def sc_select(eid_hbm, g_hbm, table_hbm, okey, oidx, oeid, osum,
                  eid2d, g_v, p_v, ukey_v, ck_v, ci_v, ce_v, s_v, sem):
        t = lax.axis_index("s") * 2 + lax.axis_index("c")

        pltpu.sync_copy(eid_hbm.at[t], eid2d)
        pltpu.sync_copy(g_hbm.at[t], g_v)

        def gblk(b, carry):
            base = b * NB
            for j in range(NB):
                pltpu.async_copy(
                    table_hbm.at[eid2d.at[base + j]],
                    p_v.at[pl.ds((base + j) * 128, 128)], sem)
            for j in range(NB):
                pltpu.make_async_copy(
                    table_hbm.at[eid2d.at[base + j]],
                    p_v.at[pl.ds((base + j) * 128, 128)], sem).wait()
            return carry
        lax.fori_loop(0, CHUNKS // NB, gblk, 0)

        IMIN = jnp.int32(-2147483648)
        def kbody(j, carry):
            for u in range(8):
                b0 = (j * 8 + u) * 16
                y = p_v[pl.ds(b0, 16)] + g_v[pl.ds(b0, 16)]
                si = lax.bitcast_convert_type(y, jnp.int32)
                sk = jnp.where(si < 0, si ^ jnp.int32(0x7FFFFFFF), si)
                valid = (b0 + lax.iota(jnp.int32, 16)) < GS
                ukey_v[pl.ds(b0, 16)] = jnp.where(valid, sk, IMIN)
            return carry
        lax.fori_loop(0, VREGS // 8, kbody, 0)

        lane15 = jnp.full((16,), 15, jnp.int32)
        def mx_body(j, acc):
            for u in range(8):
                acc = jnp.maximum(acc, ukey_v[pl.ds((j * 8 + u) * 16, 16)])
            return acc
        mvec = lax.fori_loop(0, VREGS // 8, mx_body, jnp.full((16,), IMIN, jnp.int32))
        mtot = plsc.cummax(mvec).at[lane15].get(mode="promise_in_bounds")
        msk = jnp.where(mtot < 0, mtot ^ jnp.int32(0x7FFFFFFF), mtot)
        My = lax.bitcast_convert_type(msk, jnp.float32)
        def es_body(j, acc):
            for u in range(8):
                b0 = (j * 8 + u) * 16
                y = p_v[pl.ds(b0, 16)] + g_v[pl.ds(b0, 16)]
                ev = jnp.exp(y - My)
                valid = (b0 + lax.iota(jnp.int32, 16)) < GS
                acc = acc + jnp.where(valid, ev, jnp.float32(0))
            return acc
        svec = lax.fori_loop(0, VREGS // 8, es_body, jnp.zeros((16,), jnp.float32))
        s_v[pl.ds(0, 16)] = plsc.cumsum(svec).at[lane15].get(
            mode="promise_in_bounds")

        def bitstep(bi, Tu):
            cand_u = Tu | (jnp.int32(1) << (jnp.int32(31) - bi))
            cand_s = cand_u ^ IMIN
            def cnt_body(j, cvec):
                for u in range(8):
                    m = ukey_v[pl.ds((j * 8 + u) * 16, 16)] >= cand_s
                    cvec = cvec + jnp.where(m, jnp.int32(1), jnp.int32(0))
                return cvec
            cvec = lax.fori_loop(0, VREGS // 8, cnt_body, jnp.zeros((16,), jnp.int32))
            cnt = plsc.cumsum(cvec).at[lane15].get(mode="promise_in_bounds")
            return jnp.where(cnt >= K, cand_u, Tu)
        Tu = lax.fori_loop(0, 32, bitstep, jnp.full((16,), 0, jnp.int32))
        T = Tu ^ IMIN

        def cg_body(j, cvec):
            for u in range(8):
                m = ukey_v[pl.ds((j * 8 + u) * 16, 16)] > T
                cvec = cvec + jnp.where(m, jnp.int32(1), jnp.int32(0))
            return cvec
        cgv = lax.fori_loop(0, VREGS // 8, cg_body, jnp.zeros((16,), jnp.int32))
        cg = plsc.cumsum(cgv).at[lane15].get(mode="promise_in_bounds")
        need = K - cg

        def comp_body(j, carry):
            w, teq = carry
            skv = ukey_v[pl.ds(j * 16, 16)]
            gt = skv > T
            eq = skv == T
            eqc = jnp.cumsum(eq.astype(jnp.int32))
            sel = jnp.logical_or(gt, jnp.logical_and(eq, (teq + eqc) <= need))
            idxv = j * 16 + lax.iota(jnp.int32, 16)
            eidv = eid2d[j // 8, pl.ds((j % 8) * 16, 16)]
            plsc.store_compressed(ck_v.at[pl.ds(w, 16)], skv, mask=sel)
            plsc.store_compressed(ci_v.at[pl.ds(w, 16)], idxv, mask=sel)
            plsc.store_compressed(ce_v.at[pl.ds(w, 16)], eidv, mask=sel)
            return (w + jnp.sum(sel.astype(jnp.int32)),
                    teq + jnp.sum(eq.astype(jnp.int32)))
        lax.fori_loop(0, VREGS, comp_body, (jnp.int32(0), jnp.int32(0)))

        pltpu.sync_copy(ck_v.at[pl.ds(0, K)], okey.at[t])
        pltpu.sync_copy(ci_v.at[pl.ds(0, K)], oidx.at[t])
        pltpu.sync_copy(ce_v.at[pl.ds(0, K)], oeid.at[t])
        pltpu.sync_copy(s_v, osum.at[t])

    return sc_select


_sc_select = _sc_select_build()


def _tc_sort_body(ck_ref, ci_ref, ce_ref, s_ref, oe_ref, ones_ref):
    sk = ck_ref[...]
    i = ci_ref[...]
    e = ce_ref[...]
    si = jnp.where(sk < 0, sk ^ jnp.int32(0x7FFFFFFF), sk)
    y = lax.bitcast_convert_type(si, jnp.float32)
    M = jnp.max(y, axis=1, keepdims=True)
    k = jnp.exp(y - M) / s_ref[:, 0:1]
    lane = lax.broadcasted_iota(jnp.int32, (BATCH, K), 1)
    s = 2
    while s <= K:
        d = s // 2
        while d >= 1:
            bit = (lane & d) != 0
            dirup = (lane & s) == 0
            def par(x, d=d, bit=bit):
                return jnp.where(bit, jnp.roll(x, d, axis=1),
                                 jnp.roll(x, -d, axis=1))
            pk, pi, pe = par(k), par(i), par(e)
            self_first = (k > pk) | ((k == pk) & (i < pi))
            take_self = self_first == (jnp.logical_not(bit) == dirup)
            k = jnp.where(take_self, k, pk)
            i = jnp.where(take_self, i, pi)
            e = jnp.where(take_self, e, pe)
            d //= 2
        s *= 2
    oe_ref[...] = e
    ones_ref[...] = jnp.ones_like(ones_ref)


def _tc_sort(ck, ci, ce, s):
    return pl.pallas_call(
        _tc_sort_body,
        out_shape=[
            jax.ShapeDtypeStruct((BATCH, K), jnp.int32),
            jax.ShapeDtypeStruct((BATCH * K,), jnp.float32),
        ],
    )(ck, ci, ce, s)


def kernel(inputs, edges_p, max_edges):
    edge_id = inputs[:, 1].reshape(BATCH, GS)
    eid_pad = jnp.zeros((BATCH, PAD), jnp.int32).at[:, :GS].set(edge_id)
    eid3 = eid_pad.reshape(BATCH, CHUNKS, 128)
    key42 = jax.random.key(42)
    eps = 1e-20
    gs = [-(jnp.log(-jnp.log(jax.random.uniform(
        jax.random.fold_in(key42, t), (GS,), dtype=jnp.float32) + eps) + eps))
        for t in range(BATCH)]
    g_pad = jnp.zeros((BATCH, PAD), jnp.float32).at[:, :GS].set(jnp.stack(gs))

    ck, ci, ce, ssum = _sc_select(eid3, g_pad, edges_p)
    eid_sorted, ones = _tc_sort(ck, ci, ce, ssum)

    eg = jnp.repeat(jnp.arange(BATCH, dtype=jnp.int32), K)
    outputs = jnp.stack([eg, eid_sorted.reshape(-1)], axis=1)
    return outputs, ones


# speedup vs baseline: 3.7315x; 1.1652x over previous
import functools

import jax
import jax.numpy as jnp
from jax import lax
from jax.experimental import pallas as pl
from jax.experimental.pallas import tpu as pltpu
from jax.experimental.pallas import tpu_sc as plsc

BATCH = 32
GS = 25000
K = 2048
PAD = 25088
CHUNKS = 196
VREGS = PAD // 16
NB = 14


def _sc_gather_build():
    mesh = plsc.VectorSubcoreMesh(core_axis_name="c", subcore_axis_name="s")

    @functools.partial(
        pl.kernel,
        mesh=mesh,
        compiler_params=pltpu.CompilerParams(needs_layout_passes=False),
        out_type=[jax.ShapeDtypeStruct((BATCH, PAD), jnp.float32)],
        scratch_types=[
            pltpu.VMEM((CHUNKS, 128), jnp.int32),
            pltpu.VMEM((PAD,), jnp.float32),
            pltpu.SemaphoreType.DMA,
        ],
    )
    def sc_gather(eid_hbm, table_hbm, op, eid2d, p_v, sem):
        t = lax.axis_index("s") * 2 + lax.axis_index("c")
        pltpu.sync_copy(eid_hbm.at[t], eid2d)

        def gblk(b, carry):
            base = b * NB
            for j in range(NB):
                pltpu.async_copy(
                    table_hbm.at[eid2d.at[base + j]],
                    p_v.at[pl.ds((base + j) * 128, 128)], sem)
            for j in range(NB):
                pltpu.make_async_copy(
                    table_hbm.at[eid2d.at[base + j]],
                    p_v.at[pl.ds((base + j) * 128, 128)], sem).wait()
            return carry
        lax.fori_loop(0, CHUNKS // NB, gblk, 0)
        pltpu.sync_copy(p_v, op.at[t])

    return sc_gather


_sc_gather = _sc_gather_build()


def _sc_select_build():
    mesh = plsc.VectorSubcoreMesh(core_axis_name="c", subcore_axis_name="s")

    @functools.partial(
        pl.kernel,
        mesh=mesh,
        compiler_params=pltpu.CompilerParams(needs_layout_passes=False),
        out_type=[
            jax.ShapeDtypeStruct((BATCH, K), jnp.int32),
            jax.ShapeDtypeStruct((BATCH, K), jnp.int32),
            jax.ShapeDtypeStruct((BATCH, K), jnp.int32),
            jax.ShapeDtypeStruct((BATCH, 16), jnp.float32),
        ],
        scratch_types=[
            pltpu.VMEM((CHUNKS, 128), jnp.int32),
            pltpu.VMEM((PAD,), jnp.float32),
            pltpu.VMEM((PAD,), jnp.float32),
            pltpu.VMEM((PAD,), jnp.int32),
            pltpu.VMEM((K + 16,), jnp.int32),
            pltpu.VMEM((K + 16,), jnp.int32),
            pltpu.VMEM((K + 16,), jnp.int32),
            pltpu.VMEM((16,), jnp.float32),
            pltpu.SemaphoreType.DMA,
        ],
    )
    def sc_select(eid_hbm, g_hbm, p_hbm, okey, oidx, oeid, osum,
                  eid2d, g_v, p_v, ukey_v, ck_v, ci_v, ce_v, s_v, sem):
        t = lax.axis_index("s") * 2 + lax.axis_index("c")

        pltpu.sync_copy(eid_hbm.at[t], eid2d)
        pltpu.sync_copy(g_hbm.at[t], g_v)
        pltpu.sync_copy(p_hbm.at[t], p_v)

        IMIN = jnp.int32(-2147483648)
        def kbody(j, carry):
            for u in range(8):
                b0 = (j * 8 + u) * 16
                y = p_v[pl.ds(b0, 16)] + g_v[pl.ds(b0, 16)]
                si = lax.bitcast_convert_type(y, jnp.int32)
                sk = jnp.where(si < 0, si ^ jnp.int32(0x7FFFFFFF), si)
                valid = (b0 + lax.iota(jnp.int32, 16)) < GS
                ukey_v[pl.ds(b0, 16)] = jnp.where(valid, sk, IMIN)
            return carry
        lax.fori_loop(0, VREGS // 8, kbody, 0)

        lane15 = jnp.full((16,), 15, jnp.int32)
        def mx_body(j, acc):
            for u in range(8):
                acc = jnp.maximum(acc, ukey_v[pl.ds((j * 8 + u) * 16, 16)])
            return acc
        mvec = lax.fori_loop(0, VREGS // 8, mx_body, jnp.full((16,), IMIN, jnp.int32))
        mtot = plsc.cummax(mvec).at[lane15].get(mode="promise_in_bounds")
        msk = jnp.where(mtot < 0, mtot ^ jnp.int32(0x7FFFFFFF), mtot)
        My = lax.bitcast_convert_type(msk, jnp.float32)
        def es_body(j, acc):
            for u in range(8):
                b0 = (j * 8 + u) * 16
                y = p_v[pl.ds(b0, 16)] + g_v[pl.ds(b0, 16)]
                ev = jnp.exp(y - My)
                valid = (b0 + lax.iota(jnp.int32, 16)) < GS
                acc = acc + jnp.where(valid, ev, jnp.float32(0))
            return acc
        svec = lax.fori_loop(0, VREGS // 8, es_body, jnp.zeros((16,), jnp.float32))
        s_v[pl.ds(0, 16)] = plsc.cumsum(svec).at[lane15].get(
            mode="promise_in_bounds")

        def bitstep(bi, Tu):
            cand_u = Tu | (jnp.int32(1) << (jnp.int32(31) - bi))
            cand_s = cand_u ^ IMIN
            def cnt_body(j, cvec):
                for u in range(8):
                    m = ukey_v[pl.ds((j * 8 + u) * 16, 16)] >= cand_s
                    cvec = cvec + jnp.where(m, jnp.int32(1), jnp.int32(0))
                return cvec
            cvec = lax.fori_loop(0, VREGS // 8, cnt_body, jnp.zeros((16,), jnp.int32))
            cnt = plsc.cumsum(cvec).at[lane15].get(mode="promise_in_bounds")
            return jnp.where(cnt >= K, cand_u, Tu)
        Tu = lax.fori_loop(0, 32, bitstep, jnp.full((16,), 0, jnp.int32))
        T = Tu ^ IMIN

        def cg_body(j, cvec):
            for u in range(8):
                m = ukey_v[pl.ds((j * 8 + u) * 16, 16)] > T
                cvec = cvec + jnp.where(m, jnp.int32(1), jnp.int32(0))
            return cvec
        cgv = lax.fori_loop(0, VREGS // 8, cg_body, jnp.zeros((16,), jnp.int32))
        cg = plsc.cumsum(cgv).at[lane15].get(mode="promise_in_bounds")
        need = K - cg

        def comp_body(j, carry):
            w, teq = carry
            skv = ukey_v[pl.ds(j * 16, 16)]
            gt = skv > T
            eq = skv == T
            eqc = jnp.cumsum(eq.astype(jnp.int32))
            sel = jnp.logical_or(gt, jnp.logical_and(eq, (teq + eqc) <= need))
            idxv = j * 16 + lax.iota(jnp.int32, 16)
            eidv = eid2d[j // 8, pl.ds((j % 8) * 16, 16)]
            plsc.store_compressed(ck_v.at[pl.ds(w, 16)], skv, mask=sel)
            plsc.store_compressed(ci_v.at[pl.ds(w, 16)], idxv, mask=sel)
            plsc.store_compressed(ce_v.at[pl.ds(w, 16)], eidv, mask=sel)
            return (w + jnp.sum(sel.astype(jnp.int32)),
                    teq + jnp.sum(eq.astype(jnp.int32)))
        lax.fori_loop(0, VREGS, comp_body, (jnp.int32(0), jnp.int32(0)))

        pltpu.sync_copy(ck_v.at[pl.ds(0, K)], okey.at[t])
        pltpu.sync_copy(ci_v.at[pl.ds(0, K)], oidx.at[t])
        pltpu.sync_copy(ce_v.at[pl.ds(0, K)], oeid.at[t])
        pltpu.sync_copy(s_v, osum.at[t])

    return sc_select


_sc_select = _sc_select_build()


def _tc_sort_body(ck_ref, ci_ref, ce_ref, s_ref, oe_ref, ones_ref):
    sk = ck_ref[...]
    i = ci_ref[...]
    e = ce_ref[...]
    si = jnp.where(sk < 0, sk ^ jnp.int32(0x7FFFFFFF), sk)
    y = lax.bitcast_convert_type(si, jnp.float32)
    M = jnp.max(y, axis=1, keepdims=True)
    k = jnp.exp(y - M) / s_ref[:, 0:1]
    lane = lax.broadcasted_iota(jnp.int32, (BATCH, K), 1)
    s = 2
    while s <= K:
        d = s // 2
        while d >= 1:
            bit = (lane & d) != 0
            dirup = (lane & s) == 0
            def par(x, d=d, bit=bit):
                return jnp.where(bit, jnp.roll(x, d, axis=1),
                                 jnp.roll(x, -d, axis=1))
            pk, pi, pe = par(k), par(i), par(e)
            self_first = (k > pk) | ((k == pk) & (i < pi))
            take_self = self_first == (jnp.logical_not(bit) == dirup)
            k = jnp.where(take_self, k, pk)
            i = jnp.where(take_self, i, pi)
            e = jnp.where(take_self, e, pe)
            d //= 2
        s *= 2
    oe_ref[...] = e
    ones_ref[...] = jnp.ones_like(ones_ref)


def _tc_sort(ck, ci, ce, s):
    return pl.pallas_call(
        _tc_sort_body,
        out_shape=[
            jax.ShapeDtypeStruct((BATCH, K), jnp.int32),
            jax.ShapeDtypeStruct((BATCH * K,), jnp.float32),
        ],
    )(ck, ci, ce, s)


def kernel(inputs, edges_p, max_edges):
    edge_id = inputs[:, 1].reshape(BATCH, GS)
    eid_pad = jnp.zeros((BATCH, PAD), jnp.int32).at[:, :GS].set(edge_id)
    eid3 = eid_pad.reshape(BATCH, CHUNKS, 128)
    key42 = jax.random.key(42)
    eps = 1e-20
    gs = [-(jnp.log(-jnp.log(jax.random.uniform(
        jax.random.fold_in(key42, t), (GS,), dtype=jnp.float32) + eps) + eps))
        for t in range(BATCH)]
    g_pad = jnp.zeros((BATCH, PAD), jnp.float32).at[:, :GS].set(jnp.stack(gs))

    p_rows = _sc_gather(eid3, edges_p)[0]
    ck, ci, ce, ssum = _sc_select(eid3, g_pad, p_rows)
    eid_sorted, ones = _tc_sort(ck, ci, ce, ssum)

    eg = jnp.repeat(jnp.arange(BATCH, dtype=jnp.int32), K)
    outputs = jnp.stack([eg, eid_sorted.reshape(-1)], axis=1)
    return outputs, ones
